# Initial kernel scaffold; baseline (speedup 1.0000x reference)
import jax
import jax.numpy as jnp
from jax.experimental import pallas as pl


def kernel(sentence, emb, fc_w, fc_b, out_w, out_b):
    raise NotImplementedError("write your pallas kernel here")


# speedup vs baseline: 13.8745x; 13.8745x over previous
import functools

import jax
import jax.numpy as jnp
from jax import lax
from jax.experimental import pallas as pl
from jax.experimental.pallas import tpu as pltpu
from jax.experimental.pallas import tpu_sc as plsc

_NC = 2
_NS = 16
_NW = _NC * _NS

_IDX_MINOR = 128
_IDX_MAJOR = 8
_GROUP_ROWS = _IDX_MAJOR * _IDX_MINOR


def _sc_gather(emb, idx3):
    n_groups = idx3.shape[0]
    d = emb.shape[1]
    groups_per_worker = n_groups // _NW
    mesh = plsc.VectorSubcoreMesh(core_axis_name="c", subcore_axis_name="s")

    @functools.partial(
        pl.kernel,
        out_type=jax.ShapeDtypeStruct((n_groups * _GROUP_ROWS, d), jnp.float32),
        mesh=mesh,
        scratch_types=[
            pltpu.VMEM((_IDX_MAJOR, _IDX_MINOR), jnp.int32),
            pltpu.VMEM((_GROUP_ROWS, d), jnp.float32),
            pltpu.SemaphoreType.DMA,
        ],
        compiler_params=pltpu.CompilerParams(use_tc_tiling_on_sc=False),
    )
    def gather_kernel(table_hbm, idx_hbm, out_hbm, idx_v, rows_v, sem):
        wid = lax.axis_index("s") * _NC + lax.axis_index("c")

        def body(i, carry):
            g = wid * groups_per_worker + i
            pltpu.sync_copy(idx_hbm.at[g], idx_v)
            copies = [
                pltpu.async_copy(
                    table_hbm.at[idx_v.at[b]],
                    rows_v.at[pl.ds(b * _IDX_MINOR, _IDX_MINOR)],
                    sem,
                )
                for b in range(_IDX_MAJOR)
            ]
            for c in copies:
                c.wait()
            pltpu.sync_copy(rows_v, out_hbm.at[pl.ds(g * _GROUP_ROWS, _GROUP_ROWS)])
            return carry

        lax.fori_loop(0, groups_per_worker, body, 0)

    return gather_kernel(emb, idx3)


def _tc_mlp(embeds, w1t, b1, w2t, b2):
    n, d = embeds.shape
    h = w1t.shape[1]
    tags = w2t.shape[1]
    bn = 8192
    grid = n // bn

    def mlp_kernel(e_ref, w1_ref, b1_ref, w2_ref, b2_ref, o_ref):
        e = e_ref[...]
        hid = jnp.dot(e, w1_ref[...], preferred_element_type=jnp.float32)
        hid = jnp.maximum(hid + b1_ref[...], 0.0)
        t = jnp.dot(hid, w2_ref[...], preferred_element_type=jnp.float32)
        t = t + b2_ref[...]
        m = jnp.max(t, axis=1, keepdims=True)
        s = jnp.log(jnp.sum(jnp.exp(t - m), axis=1, keepdims=True))
        o_ref[...] = t - m - s

    return pl.pallas_call(
        mlp_kernel,
        grid=(grid,),
        in_specs=[
            pl.BlockSpec((bn, d), lambda i: (i, 0)),
            pl.BlockSpec((d, h), lambda i: (0, 0)),
            pl.BlockSpec((1, h), lambda i: (0, 0)),
            pl.BlockSpec((h, tags), lambda i: (0, 0)),
            pl.BlockSpec((1, tags), lambda i: (0, 0)),
        ],
        out_specs=pl.BlockSpec((bn, tags), lambda i: (i, 0)),
        out_shape=jax.ShapeDtypeStruct((n, tags), jnp.float32),
    )(embeds, w1t, b1, w2t, b2)


def kernel(sentence, emb, fc_w, fc_b, out_w, out_b):
    n = sentence.shape[0]
    idx3 = sentence.astype(jnp.int32).reshape(
        n // _GROUP_ROWS, _IDX_MAJOR, _IDX_MINOR
    )
    embeds = _sc_gather(emb, idx3)
    return _tc_mlp(
        embeds,
        fc_w.T,
        fc_b.reshape(1, -1),
        out_w.T,
        out_b.reshape(1, -1),
    )


# speedup vs baseline: 14.6022x; 1.0524x over previous
import functools

import jax
import jax.numpy as jnp
from jax import lax
from jax.experimental import pallas as pl
from jax.experimental.pallas import tpu as pltpu
from jax.experimental.pallas import tpu_sc as plsc

_NC = 2
_NS = 16
_NW = _NC * _NS

_IDX_MINOR = 128
_IDX_MAJOR = 8
_GROUP_ROWS = _IDX_MAJOR * _IDX_MINOR

_PACK = 4
_BQ = 4096


def _sc_gather(emb, idx3):
    n_groups = idx3.shape[0]
    d = emb.shape[1]
    groups_per_worker = n_groups // _NW
    mesh = plsc.VectorSubcoreMesh(core_axis_name="c", subcore_axis_name="s")

    @functools.partial(
        pl.kernel,
        out_type=jax.ShapeDtypeStruct((n_groups * _GROUP_ROWS, d), jnp.float32),
        mesh=mesh,
        scratch_types=[
            pltpu.VMEM((_IDX_MAJOR, _IDX_MINOR), jnp.int32),
            pltpu.VMEM((_GROUP_ROWS, d), jnp.float32),
            pltpu.SemaphoreType.DMA,
        ],
        compiler_params=pltpu.CompilerParams(use_tc_tiling_on_sc=False),
    )
    def gather_kernel(table_hbm, idx_hbm, out_hbm, idx_v, rows_v, sem):
        wid = lax.axis_index("s") * _NC + lax.axis_index("c")

        def body(i, carry):
            g = wid * groups_per_worker + i
            pltpu.sync_copy(idx_hbm.at[g], idx_v)
            copies = [
                pltpu.async_copy(
                    table_hbm.at[idx_v.at[b]],
                    rows_v.at[pl.ds(b * _IDX_MINOR, _IDX_MINOR)],
                    sem,
                )
                for b in range(_IDX_MAJOR)
            ]
            for c in copies:
                c.wait()
            pltpu.sync_copy(rows_v, out_hbm.at[pl.ds(g * _GROUP_ROWS, _GROUP_ROWS)])
            return carry

        lax.fori_loop(0, groups_per_worker, body, 0)

    return gather_kernel(emb, idx3)


def _tc_mlp_packed(e4, w1k, b1k, w2k, b2k, tags):
    q, dk = e4.shape
    hk = w1k.shape[1]
    hp = hk // _PACK
    grid = q // _BQ

    def mlp_kernel(e_ref, w1_ref, b1_ref, w2_ref, b2_ref, o_ref):
        e = e_ref[...]
        hid = jnp.dot(e, w1_ref[...], preferred_element_type=jnp.float32)
        hid = jnp.maximum(hid + b1_ref[...], 0.0)
        t4 = jnp.dot(hid, w2_ref[...], preferred_element_type=jnp.float32)
        t4 = t4 + b2_ref[...]
        mask = lax.broadcasted_iota(jnp.int32, (_BQ, hp), 1) < tags
        neg_inf = jnp.float32(-jnp.inf)
        for g in range(_PACK):
            u = t4[:, g * hp:(g + 1) * hp]
            m = jnp.max(jnp.where(mask, u, neg_inf), axis=1, keepdims=True)
            ex = jnp.where(mask, jnp.exp(u - m), 0.0)
            s = jnp.log(jnp.sum(ex, axis=1, keepdims=True))
            r = u - m - s
            o_ref[g] = r[:, :tags]

    return pl.pallas_call(
        mlp_kernel,
        grid=(grid,),
        in_specs=[
            pl.BlockSpec((_BQ, dk), lambda i: (i, 0)),
            pl.BlockSpec(w1k.shape, lambda i: (0, 0)),
            pl.BlockSpec(b1k.shape, lambda i: (0, 0)),
            pl.BlockSpec(w2k.shape, lambda i: (0, 0)),
            pl.BlockSpec(b2k.shape, lambda i: (0, 0)),
        ],
        out_specs=pl.BlockSpec((_PACK, _BQ, tags), lambda i: (0, i, 0)),
        out_shape=jax.ShapeDtypeStruct((_PACK, q, tags), jnp.float32),
    )(e4, w1k, b1k, w2k, b2k)


def kernel(sentence, emb, fc_w, fc_b, out_w, out_b):
    n = sentence.shape[0]
    d = emb.shape[1]
    h = fc_w.shape[0]
    tags = out_w.shape[0]
    q = n // _PACK

    idx_p = sentence.astype(jnp.int32).reshape(_PACK, q).T.reshape(n)
    idx3 = idx_p.reshape(n // _GROUP_ROWS, _IDX_MAJOR, _IDX_MINOR)

    embeds = _sc_gather(emb, idx3)
    e4 = embeds.reshape(q, _PACK * d)

    hp = 64
    eye = jnp.eye(_PACK, dtype=jnp.float32)
    w1k = jnp.kron(eye, fc_w.T)
    b1k = jnp.tile(fc_b, _PACK).reshape(1, _PACK * h)
    w2p = jnp.pad(out_w.T, ((0, 0), (0, hp - tags)))
    w2k = jnp.kron(eye, w2p)
    b2k = jnp.tile(jnp.pad(out_b, (0, hp - tags)), _PACK).reshape(1, _PACK * hp)

    out3 = _tc_mlp_packed(e4, w1k, b1k, w2k, b2k, tags)
    return out3.reshape(n, tags)


# speedup vs baseline: 20.3723x; 1.3952x over previous
import functools

import jax
import jax.numpy as jnp
from jax import lax
from jax.experimental import pallas as pl
from jax.experimental.pallas import tpu as pltpu
from jax.experimental.pallas import tpu_sc as plsc

_NC = 2
_NS = 16
_NW = _NC * _NS

_IDX_MINOR = 128
_IDX_MAJOR = 8
_GROUP_ROWS = _IDX_MAJOR * _IDX_MINOR

_PACK = 4
_BQ = 4096


def _sc_gather(emb, idx3):
    n_groups = idx3.shape[0]
    d = emb.shape[1]
    rows_per_packed = _PACK * d
    q_group = _GROUP_ROWS // _PACK
    groups_per_worker = n_groups // _NW
    mesh = plsc.VectorSubcoreMesh(core_axis_name="c", subcore_axis_name="s")

    @functools.partial(
        pl.kernel,
        out_type=jax.ShapeDtypeStruct((n_groups * _GROUP_ROWS, d), jnp.float32),
        mesh=mesh,
        scratch_types=[
            pltpu.VMEM((_IDX_MAJOR, _IDX_MINOR), jnp.int32),
            pltpu.VMEM((_GROUP_ROWS, d), jnp.float32),
            pltpu.SemaphoreType.DMA,
        ],
        compiler_params=pltpu.CompilerParams(use_tc_tiling_on_sc=False),
    )
    def gather_kernel(table_hbm, idx_hbm, out_hbm, idx_v, rows_v, sem):
        wid = lax.axis_index("s") * _NC + lax.axis_index("c")

        def body(i, carry):
            g = wid * groups_per_worker + i
            pltpu.sync_copy(idx_hbm.at[g], idx_v)
            copies = [
                pltpu.async_copy(
                    table_hbm.at[idx_v.at[b]],
                    rows_v.at[pl.ds(b * _IDX_MINOR, _IDX_MINOR)],
                    sem,
                )
                for b in range(_IDX_MAJOR)
            ]
            for c in copies:
                c.wait()
            pltpu.sync_copy(rows_v, out_hbm.at[pl.ds(g * _GROUP_ROWS, _GROUP_ROWS)])
            return carry

        lax.fori_loop(0, groups_per_worker, body, 0)

    return gather_kernel(emb, idx3)


def _tc_mlp_packed(e4, w1k, b1k, w2k, b2k, sk, tags):
    q, dk = e4.shape
    hk = w1k.shape[1]
    hp = hk // _PACK
    grid = q // _BQ

    def mlp_kernel(e_ref, w1_ref, b1_ref, w2_ref, b2_ref, s_ref, o_ref):
        e = e_ref[...]
        hid = jnp.dot(e, w1_ref[...], preferred_element_type=jnp.float32)
        hid = jnp.maximum(hid + b1_ref[...], 0.0)
        t4 = jnp.dot(hid, w2_ref[...], preferred_element_type=jnp.float32)
        t4 = t4 + b2_ref[...]
        m = jnp.max(t4)
        ex = jnp.exp(t4 - m)
        sums = jnp.dot(ex, s_ref[...], preferred_element_type=jnp.float32)
        r = t4 - (m + jnp.log(sums))
        for g in range(_PACK):
            o_ref[g] = r[:, g * hp:g * hp + tags]

    return pl.pallas_call(
        mlp_kernel,
        grid=(grid,),
        in_specs=[
            pl.BlockSpec((_BQ, dk), lambda i: (i, 0)),
            pl.BlockSpec(w1k.shape, lambda i: (0, 0)),
            pl.BlockSpec(b1k.shape, lambda i: (0, 0)),
            pl.BlockSpec(w2k.shape, lambda i: (0, 0)),
            pl.BlockSpec(b2k.shape, lambda i: (0, 0)),
            pl.BlockSpec(sk.shape, lambda i: (0, 0)),
        ],
        out_specs=pl.BlockSpec((_PACK, _BQ, tags), lambda i: (0, i, 0)),
        out_shape=jax.ShapeDtypeStruct((_PACK, q, tags), jnp.float32),
    )(e4, w1k, b1k, w2k, b2k, sk)


def kernel(sentence, emb, fc_w, fc_b, out_w, out_b):
    n = sentence.shape[0]
    h = fc_w.shape[0]
    tags = out_w.shape[0]
    q = n // _PACK
    hp = 64

    idx_p = sentence.astype(jnp.int32).reshape(_PACK, q).T.reshape(n)
    idx3 = idx_p.reshape(n // _GROUP_ROWS, _IDX_MAJOR, _IDX_MINOR)

    embeds = _sc_gather(emb, idx3)
    e4 = embeds.reshape(q, _PACK * emb.shape[1])

    eye = jnp.eye(_PACK, dtype=jnp.float32)
    w1k = jnp.kron(eye, fc_w.T)
    b1k = jnp.tile(fc_b, _PACK).reshape(1, _PACK * h)
    w2p = jnp.pad(out_w.T, ((0, 0), (0, hp - tags)))
    w2k = jnp.kron(eye, w2p)
    b2k = jnp.tile(jnp.pad(out_b, (0, hp - tags)), _PACK).reshape(1, _PACK * hp)
    mask_ones = (jnp.arange(hp)[:, None] < tags).astype(jnp.float32)
    sk = jnp.kron(eye, jnp.broadcast_to(mask_ones, (hp, hp)))

    out3 = _tc_mlp_packed(e4, w1k, b1k, w2k, b2k, sk, tags)
    return out3.reshape(n, tags)
